# baseline (speedup 1.0000x reference)
import functools

import jax
import jax.numpy as jnp
from jax import lax
from jax.experimental import pallas as pl
from jax.experimental.pallas import tpu as pltpu
from jax.experimental.pallas import tpu_sc as plsc

BATCH = 16384
DIM = 32

_INFO = plsc.get_sparse_core_info()
_NC = _INFO.num_cores
_NS = _INFO.num_subcores
_NW = _NC * _NS
_BPW = BATCH // _NW
_CHUNK = 128
_NCHUNK = _BPW // _CHUNK


def _gather3_body(u_idx, p_idx, n_idx, uw, iw,
                  u_out, p_out, n_out,
                  idx_u, idx_p, idx_n,
                  rows_u, rows_p, rows_n,
                  sem_u, sem_p, sem_n, sem_s):
    wid = lax.axis_index("s") * _NC + lax.axis_index("c")
    base = wid * _BPW
    cbase = wid * _NCHUNK

    pltpu.sync_copy(u_idx.at[pl.ds(cbase, _NCHUNK)], idx_u)
    pltpu.sync_copy(p_idx.at[pl.ds(cbase, _NCHUNK)], idx_p)
    pltpu.sync_copy(n_idx.at[pl.ds(cbase, _NCHUNK)], idx_n)

    gathers = []
    for idx, table, rows, sem in ((idx_u, uw, rows_u, sem_u),
                                  (idx_p, iw, rows_p, sem_p),
                                  (idx_n, iw, rows_n, sem_n)):
        for j in range(_NCHUNK):
            gathers.append(
                pltpu.async_copy(table.at[idx.at[j]],
                                 rows.at[pl.ds(j * _CHUNK, _CHUNK)], sem))

    stores = []
    for g in range(3):
        for j in range(_NCHUNK):
            gathers[g * _NCHUNK + j].wait()
        rows, out = ((rows_u, u_out), (rows_p, p_out), (rows_n, n_out))[g]
        stores.append(
            pltpu.async_copy(rows, out.at[pl.ds(base, _BPW)], sem_s))
    for s in stores:
        s.wait()


@jax.jit
def _gather3(u_idx, p_idx, n_idx, uw, iw):
    out_ty = jax.ShapeDtypeStruct((BATCH, DIM), jnp.float32)
    run = pl.kernel(
        _gather3_body,
        mesh=plsc.VectorSubcoreMesh(core_axis_name="c", subcore_axis_name="s"),
        compiler_params=pltpu.CompilerParams(use_tc_tiling_on_sc=False),
        out_type=(out_ty, out_ty, out_ty),
        scratch_types=[
            pltpu.VMEM((_NCHUNK, _CHUNK), jnp.int32),
            pltpu.VMEM((_NCHUNK, _CHUNK), jnp.int32),
            pltpu.VMEM((_NCHUNK, _CHUNK), jnp.int32),
            pltpu.VMEM((_BPW, DIM), jnp.float32),
            pltpu.VMEM((_BPW, DIM), jnp.float32),
            pltpu.VMEM((_BPW, DIM), jnp.float32),
            pltpu.SemaphoreType.DMA,
            pltpu.SemaphoreType.DMA,
            pltpu.SemaphoreType.DMA,
            pltpu.SemaphoreType.DMA,
        ],
    )
    return run(u_idx, p_idx, n_idx, uw, iw)


def kernel(users, pos_items, neg_items, user_weight, item_weight):
    u = users.astype(jnp.int32).reshape(_NW * _NCHUNK, _CHUNK)
    p = pos_items.astype(jnp.int32).reshape(_NW * _NCHUNK, _CHUNK)
    n = neg_items.astype(jnp.int32).reshape(_NW * _NCHUNK, _CHUNK)
    return _gather3(u, p, n, user_weight, item_weight)
